# Initial kernel scaffold; baseline (speedup 1.0000x reference)
import jax
import jax.numpy as jnp
from jax.experimental import pallas as pl


def kernel(x, W):
    raise NotImplementedError("write your pallas kernel here")


# speedup vs baseline: 1.3214x; 1.3214x over previous
import jax
import jax.numpy as jnp
from jax import lax
from jax.experimental import pallas as pl
from jax.experimental.pallas import tpu as pltpu

_B, _T, _D, _E, _TOPK = 4, 4096, 4096, 64, 2
_TM = 512


def _router_block(x_ref, w_ref, probs_ref, idx_ref, wts_ref):
    x = x_ref[...]
    w = w_ref[...]
    logits = lax.dot_general(x, w, (((1,), (1,)), ((), ())),
                             preferred_element_type=jnp.float32)
    m = jnp.max(logits, axis=-1, keepdims=True)
    ex = jnp.exp(logits - m)
    probs = ex / jnp.sum(ex, axis=-1, keepdims=True)
    probs_ref[...] = probs

    lane = lax.broadcasted_iota(jnp.int32, probs.shape, 1)
    m1 = jnp.max(probs, axis=-1, keepdims=True)
    i1 = jnp.min(jnp.where(probs == m1, lane, _E), axis=-1, keepdims=True)
    masked = jnp.where(lane == i1, -1.0, probs)
    m2 = jnp.max(masked, axis=-1, keepdims=True)
    i2 = jnp.min(jnp.where(masked == m2, lane, _E), axis=-1, keepdims=True)
    s = m1 + m2
    idx_ref[:, 0:1] = i1
    idx_ref[:, 1:2] = i2
    wts_ref[:, 0:1] = m1 / s
    wts_ref[:, 1:2] = m2 / s


def kernel(x, W):
    BT = _B * _T
    x2 = x.reshape(BT, _D)
    grid = (BT // _TM,)
    probs, idx, wts = pl.pallas_call(
        _router_block,
        grid=grid,
        in_specs=[
            pl.BlockSpec((_TM, _D), lambda i: (i, 0)),
            pl.BlockSpec((_E, _D), lambda i: (0, 0)),
        ],
        out_specs=[
            pl.BlockSpec((_TM, _E), lambda i: (i, 0)),
            pl.BlockSpec((_TM, _TOPK), lambda i: (i, 0)),
            pl.BlockSpec((_TM, _TOPK), lambda i: (i, 0)),
        ],
        out_shape=[
            jax.ShapeDtypeStruct((BT, _E), jnp.float32),
            jax.ShapeDtypeStruct((BT, _TOPK), jnp.int32),
            jax.ShapeDtypeStruct((BT, _TOPK), jnp.float32),
        ],
    )(x2, W)
    return (probs.reshape(_B, _T, _E),
            idx.reshape(_B, _T, _TOPK),
            wts.reshape(_B, _T, _TOPK))


# speedup vs baseline: 1.4060x; 1.0640x over previous
import jax
import jax.numpy as jnp
from jax import lax
from jax.experimental import pallas as pl
from jax.experimental.pallas import tpu as pltpu

_B, _T, _D, _E, _TOPK = 4, 4096, 4096, 64, 2
_TM = 1024


def _router_block(x_ref, w_ref, probs_ref, idx_ref, wts_ref):
    x = x_ref[...]
    w = w_ref[...]
    logits = lax.dot_general(x, w, (((1,), (1,)), ((), ())),
                             preferred_element_type=jnp.float32)
    m = jnp.max(logits, axis=-1, keepdims=True)
    ex = jnp.exp(logits - m)
    probs = ex / jnp.sum(ex, axis=-1, keepdims=True)
    probs_ref[...] = probs

    lane = lax.broadcasted_iota(jnp.int32, probs.shape, 1)
    m1 = jnp.max(probs, axis=-1, keepdims=True)
    i1 = jnp.min(jnp.where(probs == m1, lane, _E), axis=-1, keepdims=True)
    masked = jnp.where(lane == i1, -1.0, probs)
    m2 = jnp.max(masked, axis=-1, keepdims=True)
    i2 = jnp.min(jnp.where(masked == m2, lane, _E), axis=-1, keepdims=True)
    s = m1 + m2
    idx_ref[:, 0:1] = i1
    idx_ref[:, 1:2] = i2
    wts_ref[:, 0:1] = m1 / s
    wts_ref[:, 1:2] = m2 / s


def kernel(x, W):
    BT = _B * _T
    x2 = x.reshape(BT, _D)
    grid = (BT // _TM,)
    probs, idx, wts = pl.pallas_call(
        _router_block,
        grid=grid,
        in_specs=[
            pl.BlockSpec((_TM, _D), lambda i: (i, 0)),
            pl.BlockSpec((_E, _D), lambda i: (0, 0)),
        ],
        out_specs=[
            pl.BlockSpec((_TM, _E), lambda i: (i, 0)),
            pl.BlockSpec((_TM, _TOPK), lambda i: (i, 0)),
            pl.BlockSpec((_TM, _TOPK), lambda i: (i, 0)),
        ],
        out_shape=[
            jax.ShapeDtypeStruct((BT, _E), jnp.float32),
            jax.ShapeDtypeStruct((BT, _TOPK), jnp.int32),
            jax.ShapeDtypeStruct((BT, _TOPK), jnp.float32),
        ],
    )(x2, W)
    return (probs.reshape(_B, _T, _E),
            idx.reshape(_B, _T, _TOPK),
            wts.reshape(_B, _T, _TOPK))


# speedup vs baseline: 1.4079x; 1.0014x over previous
import jax
import jax.numpy as jnp
from jax import lax
from jax.experimental import pallas as pl
from jax.experimental.pallas import tpu as pltpu

_B, _T, _D, _E, _TOPK = 4, 4096, 4096, 64, 2
_TM = 1024


def _router_block(x_ref, w_ref, probs_ref, idx_ref, wts_ref):
    x = x_ref[...]
    w = w_ref[...]
    logits = lax.dot_general(x, w, (((1,), (1,)), ((), ())),
                             preferred_element_type=jnp.float32)
    m = jnp.max(logits, axis=-1, keepdims=True)
    ex = jnp.exp(logits - m)
    probs = ex / jnp.sum(ex, axis=-1, keepdims=True)
    probs_ref[...] = probs

    lane = lax.broadcasted_iota(jnp.int32, probs.shape, 1)
    m1 = jnp.max(probs, axis=-1, keepdims=True)
    i1 = jnp.min(jnp.where(probs == m1, lane, _E), axis=-1, keepdims=True)
    masked = jnp.where(lane == i1, -1.0, probs)
    m2 = jnp.max(masked, axis=-1, keepdims=True)
    i2 = jnp.min(jnp.where(masked == m2, lane, _E), axis=-1, keepdims=True)
    s = m1 + m2
    idx_ref[:, 0:1] = i1
    idx_ref[:, 1:2] = i2
    wts_ref[:, 0:1] = m1 / s
    wts_ref[:, 1:2] = m2 / s


def kernel(x, W):
    BT = _B * _T
    x2 = x.reshape(BT, _D)
    grid = (BT // _TM,)
    probs, idx, wts = pl.pallas_call(
        _router_block,
        grid=grid,
        in_specs=[
            pl.BlockSpec((_TM, _D), lambda i: (i, 0)),
            pl.BlockSpec((_E, _D), lambda i: (0, 0)),
        ],
        out_specs=[
            pl.BlockSpec((_TM, _E), lambda i: (i, 0)),
            pl.BlockSpec((_TM, _TOPK), lambda i: (i, 0)),
            pl.BlockSpec((_TM, _TOPK), lambda i: (i, 0)),
        ],
        out_shape=[
            jax.ShapeDtypeStruct((BT, _E), jnp.float32),
            jax.ShapeDtypeStruct((BT, _TOPK), jnp.int32),
            jax.ShapeDtypeStruct((BT, _TOPK), jnp.float32),
        ],
        compiler_params=pltpu.CompilerParams(
            vmem_limit_bytes=128 * 1024 * 1024),
    )(x2, W)
    return (probs.reshape(_B, _T, _E),
            idx.reshape(_B, _T, _TOPK),
            wts.reshape(_B, _T, _TOPK))
